# baseline (speedup 1.0000x reference)
import functools

import jax
import jax.numpy as jnp
from jax import lax
from jax.experimental import pallas as pl
from jax.experimental.pallas import tpu as pltpu
from jax.experimental.pallas import tpu_sc as plsc

_BATCH = 4
_SEQ = 8192
_D = 768
_TOTAL = _BATCH * _SEQ * _D
_PER_BATCH = _SEQ * _D
_NW = 32
_SPAN = _TOTAL // _NW
_C = 16384
_N_CHUNKS = _SPAN // _C
_L = 16


def _sc_body(x_hbm, t_hbm, out_hbm, bufx, buft):
    wid = lax.axis_index("s") * 2 + lax.axis_index("c")
    xb = wid * _SPAN
    tb = lax.rem(xb, _PER_BATCH)

    def chunk_body(g, carry):
        off = g * _C
        pltpu.sync_copy(x_hbm.at[pl.ds(xb + off, _C)], bufx)
        pltpu.sync_copy(t_hbm.at[pl.ds(tb + off, _C)], buft)

        def add_body(j, c):
            v = buft[pl.ds(j * _L, _L)]
            plsc.addupdate(bufx.at[pl.ds(j * _L, _L)], v)
            return c

        lax.fori_loop(0, _C // _L, add_body, 0, unroll=8)
        pltpu.sync_copy(bufx, out_hbm.at[pl.ds(xb + off, _C)])
        return carry

    lax.fori_loop(0, _N_CHUNKS, chunk_body, 0)


_sc_add = functools.partial(
    pl.kernel,
    out_type=jax.ShapeDtypeStruct((_TOTAL,), jnp.float32),
    mesh=plsc.VectorSubcoreMesh(core_axis_name="c", subcore_axis_name="s"),
    scratch_types=[
        pltpu.VMEM((_C,), jnp.float32),
        pltpu.VMEM((_C,), jnp.float32),
    ],
)(_sc_body)


def kernel(x, pos_table):
    batch, seq, d = x.shape
    out_flat = _sc_add(x.reshape(-1), pos_table.reshape(-1))
    return out_flat.reshape(batch, seq, d)


# speedup vs baseline: 1.3226x; 1.3226x over previous
import functools

import jax
import jax.numpy as jnp
from jax import lax
from jax.experimental import pallas as pl
from jax.experimental.pallas import tpu as pltpu
from jax.experimental.pallas import tpu_sc as plsc

_BATCH = 4
_SEQ = 8192
_D = 768
_TOTAL = _BATCH * _SEQ * _D
_PER_BATCH = _SEQ * _D
_NW = 32
_SPAN = _TOTAL // _NW
_C = 24576
_N_CHUNKS = _SPAN // _C
_L = 16


def _sc_body(x_hbm, t_hbm, out_hbm,
             bufx0, bufx1, buft0, buft1, sin0, sin1, sout0, sout1):
    bufx = (bufx0, bufx1)
    buft = (buft0, buft1)
    sin = (sin0, sin1)
    sout = (sout0, sout1)

    wid = lax.axis_index("s") * 2 + lax.axis_index("c")
    xb = wid * _SPAN
    tb = lax.rem(xb, _PER_BATCH)

    def in_start(g, b):
        off = g * _C
        pltpu.make_async_copy(x_hbm.at[pl.ds(xb + off, _C)], bufx[b], sin[b]).start()
        pltpu.make_async_copy(t_hbm.at[pl.ds(tb + off, _C)], buft[b], sin[b]).start()

    def in_wait(g, b):
        off = g * _C
        pltpu.make_async_copy(x_hbm.at[pl.ds(xb + off, _C)], bufx[b], sin[b]).wait()
        pltpu.make_async_copy(t_hbm.at[pl.ds(tb + off, _C)], buft[b], sin[b]).wait()

    def out_start(g, b):
        pltpu.make_async_copy(bufx[b], out_hbm.at[pl.ds(xb + g * _C, _C)], sout[b]).start()

    def out_wait(g, b):
        pltpu.make_async_copy(bufx[b], out_hbm.at[pl.ds(xb + g * _C, _C)], sout[b]).wait()

    def accumulate(b):
        def add_body(j, c):
            v = buft[b][pl.ds(j * _L, _L)]
            plsc.addupdate(bufx[b].at[pl.ds(j * _L, _L)], v)
            return c

        lax.fori_loop(0, _C // _L, add_body, 0, unroll=8)

    in_start(0, 0)

    def step(i, carry):
        for b in (0, 1):
            g = 2 * i + b
            nb = 1 - b

            @pl.when(g >= 1)
            def _():
                out_wait(g - 1, nb)

            @pl.when(g + 1 < _N_CHUNKS)
            def _():
                in_start(g + 1, nb)

            in_wait(g, b)
            accumulate(b)
            out_start(g, b)
        return carry

    lax.fori_loop(0, _N_CHUNKS // 2, step, 0)
    out_wait(_N_CHUNKS - 1, (_N_CHUNKS - 1) % 2)


_sc_add = functools.partial(
    pl.kernel,
    out_type=jax.ShapeDtypeStruct((_TOTAL,), jnp.float32),
    mesh=plsc.VectorSubcoreMesh(core_axis_name="c", subcore_axis_name="s"),
    scratch_types=[
        pltpu.VMEM((_C,), jnp.float32),
        pltpu.VMEM((_C,), jnp.float32),
        pltpu.VMEM((_C,), jnp.float32),
        pltpu.VMEM((_C,), jnp.float32),
        pltpu.SemaphoreType.DMA,
        pltpu.SemaphoreType.DMA,
        pltpu.SemaphoreType.DMA,
        pltpu.SemaphoreType.DMA,
    ],
)(_sc_body)


def kernel(x, pos_table):
    batch, seq, d = x.shape
    out_flat = _sc_add(x.reshape(-1), pos_table.reshape(-1))
    return out_flat.reshape(batch, seq, d)


# speedup vs baseline: 3.3006x; 2.4954x over previous
import functools

import jax
import jax.numpy as jnp
from jax import lax
from jax.experimental import pallas as pl
from jax.experimental.pallas import tpu as pltpu
from jax.experimental.pallas import tpu_sc as plsc

_BATCH = 4
_SEQ = 8192
_D = 768
_NW = 32
_WPB = _NW // _BATCH
_ROWS_PER_W = _SEQ // _WPB
_R = 32
_N_CHUNKS = _ROWS_PER_W // _R
_L = 16


def _sc_body(x_hbm, t_hbm, out_hbm,
             bufx0, bufx1, buft0, buft1, sin0, sin1, sout0, sout1):
    bufx = (bufx0, bufx1)
    buft = (buft0, buft1)
    sin = (sin0, sin1)
    sout = (sout0, sout1)

    wid = lax.axis_index("s") * 2 + lax.axis_index("c")
    bidx = wid // _WPB
    r0 = (wid % _WPB) * _ROWS_PER_W

    def in_start(g, b):
        row = r0 + g * _R
        pltpu.make_async_copy(
            x_hbm.at[bidx, pl.ds(row, _R), :], bufx[b], sin[b]).start()
        pltpu.make_async_copy(
            t_hbm.at[pl.ds(row, _R), :], buft[b], sin[b]).start()

    def in_wait(g, b):
        row = r0 + g * _R
        pltpu.make_async_copy(
            x_hbm.at[bidx, pl.ds(row, _R), :], bufx[b], sin[b]).wait()
        pltpu.make_async_copy(
            t_hbm.at[pl.ds(row, _R), :], buft[b], sin[b]).wait()

    def out_start(g, b):
        row = r0 + g * _R
        pltpu.make_async_copy(
            bufx[b], out_hbm.at[bidx, pl.ds(row, _R), :], sout[b]).start()

    def out_wait(g, b):
        row = r0 + g * _R
        pltpu.make_async_copy(
            bufx[b], out_hbm.at[bidx, pl.ds(row, _R), :], sout[b]).wait()

    def accumulate(b):
        def row_body(r, c):
            for j in range(_D // _L):
                v = buft[b][r, pl.ds(j * _L, _L)]
                plsc.addupdate(bufx[b].at[r, pl.ds(j * _L, _L)], v)
            return c

        lax.fori_loop(0, _R, row_body, 0)

    in_start(0, 0)

    def step(i, carry):
        for b in (0, 1):
            g = 2 * i + b
            nb = 1 - b

            @pl.when(g >= 1)
            def _():
                out_wait(g - 1, nb)

            @pl.when(g + 1 < _N_CHUNKS)
            def _():
                in_start(g + 1, nb)

            in_wait(g, b)
            accumulate(b)
            out_start(g, b)
        return carry

    lax.fori_loop(0, _N_CHUNKS // 2, step, 0)
    out_wait(_N_CHUNKS - 1, (_N_CHUNKS - 1) % 2)


_sc_add = functools.partial(
    pl.kernel,
    out_type=jax.ShapeDtypeStruct((_BATCH, _SEQ, _D), jnp.float32),
    mesh=plsc.VectorSubcoreMesh(core_axis_name="c", subcore_axis_name="s"),
    compiler_params=pltpu.CompilerParams(use_tc_tiling_on_sc=True),
    scratch_types=[
        pltpu.VMEM((_R, _D), jnp.float32),
        pltpu.VMEM((_R, _D), jnp.float32),
        pltpu.VMEM((_R, _D), jnp.float32),
        pltpu.VMEM((_R, _D), jnp.float32),
        pltpu.SemaphoreType.DMA,
        pltpu.SemaphoreType.DMA,
        pltpu.SemaphoreType.DMA,
        pltpu.SemaphoreType.DMA,
    ],
)(_sc_body)


def kernel(x, pos_table):
    return _sc_add(x, pos_table)


# speedup vs baseline: 3.4905x; 1.0575x over previous
import functools

import jax
import jax.numpy as jnp
from jax import lax
from jax.experimental import pallas as pl
from jax.experimental.pallas import tpu as pltpu
from jax.experimental.pallas import tpu_sc as plsc

_BATCH = 4
_SEQ = 8192
_D = 768
_NW = 32
_ROWS_PER_W = _SEQ // _NW
_R = 32
_N_CHUNKS = _ROWS_PER_W // _R
_L = 16


def _sc_body(x_hbm, t_hbm, out_hbm,
             bufx0, bufx1, buft0, buft1, sx0, sx1, st0, st1, so0, so1):
    bufx = (bufx0, bufx1)
    buft = (buft0, buft1)
    sx = (sx0, sx1)
    st = (st0, st1)
    so = (so0, so1)

    wid = lax.axis_index("s") * 2 + lax.axis_index("c")
    r0 = wid * _ROWS_PER_W

    def x_copy(g, bb, xs):
        row = r0 + g * _R
        return pltpu.make_async_copy(
            x_hbm.at[bb, pl.ds(row, _R), :], bufx[xs], sx[xs])

    def t_copy(g, ts):
        row = r0 + g * _R
        return pltpu.make_async_copy(
            t_hbm.at[pl.ds(row, _R), :], buft[ts], st[ts])

    def out_copy(g, bb, xs):
        row = r0 + g * _R
        return pltpu.make_async_copy(
            bufx[xs], out_hbm.at[bb, pl.ds(row, _R), :], so[xs])

    def accumulate(xs, ts):
        def row_body(r, c):
            for j in range(_D // _L):
                v = buft[ts][r, pl.ds(j * _L, _L)]
                plsc.addupdate(bufx[xs].at[r, pl.ds(j * _L, _L)], v)
            return c

        lax.fori_loop(0, _R, row_body, 0)

    t_copy(0, 0).start()
    x_copy(0, 0, 0).start()

    def chunk(i, carry):
        for gp in (0, 1):
            g = 2 * i + gp

            @pl.when(g + 1 < _N_CHUNKS)
            def _():
                t_copy(g + 1, 1 - gp).start()

            t_copy(g, gp).wait()

            for bb in range(_BATCH):
                xs = bb & 1
                ns = 1 - xs
                if bb == 0:

                    @pl.when(g >= 1)
                    def _():
                        out_copy(g - 1, 3, ns).wait()

                else:
                    out_copy(g, bb - 1, ns).wait()

                if bb < _BATCH - 1:
                    x_copy(g, bb + 1, ns).start()
                else:

                    @pl.when(g + 1 < _N_CHUNKS)
                    def _():
                        x_copy(g + 1, 0, ns).start()

                x_copy(g, bb, xs).wait()
                accumulate(xs, gp)
                out_copy(g, bb, xs).start()
        return carry

    lax.fori_loop(0, _N_CHUNKS // 2, chunk, 0)
    out_copy(_N_CHUNKS - 1, 3, 1).wait()


_sc_add = functools.partial(
    pl.kernel,
    out_type=jax.ShapeDtypeStruct((_BATCH, _SEQ, _D), jnp.float32),
    mesh=plsc.VectorSubcoreMesh(core_axis_name="c", subcore_axis_name="s"),
    compiler_params=pltpu.CompilerParams(use_tc_tiling_on_sc=True),
    scratch_types=[
        pltpu.VMEM((_R, _D), jnp.float32),
        pltpu.VMEM((_R, _D), jnp.float32),
        pltpu.VMEM((_R, _D), jnp.float32),
        pltpu.VMEM((_R, _D), jnp.float32),
        pltpu.SemaphoreType.DMA,
        pltpu.SemaphoreType.DMA,
        pltpu.SemaphoreType.DMA,
        pltpu.SemaphoreType.DMA,
        pltpu.SemaphoreType.DMA,
        pltpu.SemaphoreType.DMA,
    ],
)(_sc_body)


def kernel(x, pos_table):
    return _sc_add(x, pos_table)


# speedup vs baseline: 3.9779x; 1.1396x over previous
import functools

import jax
import jax.numpy as jnp
from jax import lax
from jax.experimental import pallas as pl
from jax.experimental.pallas import tpu as pltpu
from jax.experimental.pallas import tpu_sc as plsc

_BATCH = 4
_SEQ = 8192
_D = 768
_NW = 32
_ROWS_PER_W = _SEQ // _NW
_R = 32
_N_CHUNKS = _ROWS_PER_W // _R
_L = 16


def _sc_body(x_hbm, t_hbm, out_hbm,
             bufx0, bufx1, buft0, buft1, sx0, sx1, st0, st1, so0, so1):
    bufx = (bufx0, bufx1)
    buft = (buft0, buft1)
    sx = (sx0, sx1)
    st = (st0, st1)
    so = (so0, so1)

    wid = lax.axis_index("s") * 2 + lax.axis_index("c")
    r0 = wid * _ROWS_PER_W

    def x_copy(g, bb, xs):
        row = r0 + g * _R
        return pltpu.make_async_copy(
            x_hbm.at[bb, pl.ds(row, _R), :], bufx[xs], sx[xs])

    def t_copy(g, ts):
        row = r0 + g * _R
        return pltpu.make_async_copy(
            t_hbm.at[pl.ds(row, _R), :], buft[ts], st[ts])

    def out_copy(g, bb, xs):
        row = r0 + g * _R
        return pltpu.make_async_copy(
            bufx[xs], out_hbm.at[bb, pl.ds(row, _R), :], so[xs])

    def accumulate(xs, ts):
        @plsc.parallel_loop(0, _R, 1)
        def _(r):
            vs = [buft[ts][r, pl.ds(j * _L, _L)] for j in range(_D // _L)]
            for j, v in enumerate(vs):
                plsc.addupdate(bufx[xs].at[r, pl.ds(j * _L, _L)], v)

    t_copy(0, 0).start()
    x_copy(0, 0, 0).start()

    def chunk(i, carry):
        for gp in (0, 1):
            g = 2 * i + gp

            @pl.when(g + 1 < _N_CHUNKS)
            def _():
                t_copy(g + 1, 1 - gp).start()

            t_copy(g, gp).wait()

            for bb in range(_BATCH):
                xs = bb & 1
                ns = 1 - xs
                if bb == 0:

                    @pl.when(g >= 1)
                    def _():
                        out_copy(g - 1, 3, ns).wait()

                else:
                    out_copy(g, bb - 1, ns).wait()

                if bb < _BATCH - 1:
                    x_copy(g, bb + 1, ns).start()
                else:

                    @pl.when(g + 1 < _N_CHUNKS)
                    def _():
                        x_copy(g + 1, 0, ns).start()

                x_copy(g, bb, xs).wait()
                accumulate(xs, gp)
                out_copy(g, bb, xs).start()
        return carry

    lax.fori_loop(0, _N_CHUNKS // 2, chunk, 0)
    out_copy(_N_CHUNKS - 1, 3, 1).wait()


_sc_add = functools.partial(
    pl.kernel,
    out_type=jax.ShapeDtypeStruct((_BATCH, _SEQ, _D), jnp.float32),
    mesh=plsc.VectorSubcoreMesh(core_axis_name="c", subcore_axis_name="s"),
    compiler_params=pltpu.CompilerParams(use_tc_tiling_on_sc=True),
    scratch_types=[
        pltpu.VMEM((_R, _D), jnp.float32),
        pltpu.VMEM((_R, _D), jnp.float32),
        pltpu.VMEM((_R, _D), jnp.float32),
        pltpu.VMEM((_R, _D), jnp.float32),
        pltpu.SemaphoreType.DMA,
        pltpu.SemaphoreType.DMA,
        pltpu.SemaphoreType.DMA,
        pltpu.SemaphoreType.DMA,
        pltpu.SemaphoreType.DMA,
        pltpu.SemaphoreType.DMA,
    ],
)(_sc_body)


def kernel(x, pos_table):
    return _sc_add(x, pos_table)
